# baseline (speedup 1.0000x reference)
import functools

import jax
import jax.numpy as jnp
from jax import lax
from jax.experimental import pallas as pl
from jax.experimental.pallas import tpu as pltpu
from jax.experimental.pallas import tpu_sc as plsc

N = 10000
NP = 10240
E = 320000
D = 64
NT = 16
EPT = E // NT
G = 128
NG = 160
EPT_PAD = NG * G
R2D = NT * NG
NSLC = NP // NT

f32 = jnp.float32
i32 = jnp.int32

_MESH = plsc.VectorSubcoreMesh(core_axis_name="c", subcore_axis_name="s")
_PARAMS = pltpu.CompilerParams(needs_layout_passes=False,
                               use_tc_tiling_on_sc=False)


def _zero_buf(buf, width):
    zero16 = jnp.zeros((16,), f32)

    def body(i, c):
        for q in range(width // 16):
            buf[i, pl.ds(q * 16, 16)] = zero16
        return c

    lax.fori_loop(0, G, body, 0)


def _zero_acc(acc, buf, ns, width):
    _zero_buf(buf, width)

    def body(kk, c):
        pltpu.sync_copy(buf, acc.at[pl.ds(ns + kk * G, G)])
        return c

    lax.fori_loop(0, NSLC // G, body, 0)


def _scale(gbuf, sbuf, nwbuf, j, width):
    def ebody(e, c):
        bc = plsc.load_gather(
            nwbuf, [jnp.full((16,), j, i32), jnp.full((16,), e, i32)])
        for q in range(width // 16):
            sl = pl.ds(q * 16, 16)
            sbuf[e, sl] = gbuf[e, sl] * bc
        return c

    lax.fori_loop(0, G, ebody, 0)


def _hop(table_ref, acc, rowbuf, colbuf, nwbuf, bufs, gsem, ssem, width,
         async_scatter=True):
    g0, g1, s0, s1 = bufs
    gb = (g0, g1)
    sb = (s0, s1)

    pltpu.async_copy(table_ref.at[rowbuf.at[0]], g0, gsem)
    pltpu.async_copy(table_ref.at[rowbuf.at[1]], g1, gsem)

    def ibody(i, c):
        for p in range(2):
            j = i * 2 + p
            pltpu.make_async_copy(
                table_ref.at[rowbuf.at[j]], gb[p], gsem).wait()

            if async_scatter:
                @pl.when(j >= 2)
                def _():
                    pltpu.make_async_copy(
                        table_ref.at[pl.ds(0, G)], sb[p], ssem).wait()

                _scale(gb[p], sb[p], nwbuf, j, width)
                pltpu.async_copy(sb[p], acc.at[colbuf.at[j]], ssem, add=True)
            else:
                _scale(gb[p], gb[p], nwbuf, j, width)
                pltpu.sync_copy(gb[p], acc.at[colbuf.at[j]], add=True)

            @pl.when(j + 2 < NG)
            def _():
                pltpu.async_copy(table_ref.at[rowbuf.at[j + 2]], gb[p], gsem)
        return c

    lax.fori_loop(0, NG // 2, ibody, 0)
    if async_scatter:
        for p in range(2):
            pltpu.make_async_copy(
                table_ref.at[pl.ds(0, G)], sb[p], ssem).wait()


def _sc_rz(xh2, row2, col2, ew2):
    @functools.partial(
        pl.kernel,
        out_type=[jax.ShapeDtypeStruct((2 * NP, D), f32)] * 2
        + [jax.ShapeDtypeStruct((R2D, G), f32)],
        mesh=_MESH,
        scratch_types=[
            pltpu.VMEM((NG, G), i32),
            pltpu.VMEM((NG, G), i32),
            pltpu.VMEM((NG, G), f32),
            pltpu.VMEM((NP,), f32),
            pltpu.VMEM((G, D), f32),
            pltpu.VMEM((G, D), f32),
            pltpu.VMEM((G, D), f32),
            pltpu.VMEM((G, D), f32),
            pltpu.VMEM((G,), f32),
            pltpu.VMEM_SHARED((NP,), f32),
            pltpu.VMEM_SHARED((NP, D), f32),
            pltpu.SemaphoreType.DMA,
            pltpu.SemaphoreType.DMA,
        ],
        compiler_params=_PARAMS,
    )
    def k(xh_ref, row_ref, col_ref, ew_ref,
          y1, y2, nw_out,
          rowbuf, colbuf, nwbuf, degbuf, g0, g1, s0, s1, onesbuf,
          deg_acc, acc1, gsem, ssem):
        bufs = (g0, g1, s0, s1)
        core = lax.axis_index("c")
        tile = lax.axis_index("s")
        tb = tile * NG
        ns = tile * NSLC
        zero16 = jnp.zeros((16,), f32)

        pltpu.sync_copy(row_ref.at[pl.ds(tb, NG)], rowbuf)
        pltpu.sync_copy(col_ref.at[pl.ds(tb, NG)], colbuf)
        pltpu.sync_copy(ew_ref.at[pl.ds(tb, NG)], nwbuf)

        _zero_acc(acc1, g0, ns, D)

        def zdeg(i, c):
            degbuf[pl.ds(ns + i * 16, 16)] = zero16
            return c

        lax.fori_loop(0, NSLC // 16, zdeg, 0)
        pltpu.sync_copy(degbuf.at[pl.ds(ns, NSLC)], deg_acc.at[pl.ds(ns, NSLC)])

        def ones_(i, c):
            onesbuf[pl.ds(i * 16, 16)] = jnp.ones((16,), f32)
            return c

        lax.fori_loop(0, G // 16, ones_, 0)
        plsc.subcore_barrier()

        def dbody(j, c):
            pltpu.sync_copy(onesbuf, deg_acc.at[rowbuf.at[j]], add=True)
            return c

        lax.fori_loop(0, NG, dbody, 0)
        plsc.subcore_barrier()

        pltpu.sync_copy(deg_acc, degbuf)
        roff = core * NP

        def nbody(i, c):
            j = i // 8
            q = i % 8
            sl = pl.ds(q * 16, 16)
            idx = rowbuf[j, sl]
            dg = jnp.maximum(plsc.load_gather(degbuf, [idx]), 1.0)
            nwbuf[j, sl] = nwbuf[j, sl] / dg
            rowbuf[j, sl] = idx + roff
            return c

        lax.fori_loop(0, NG * 8, nbody, 0)

        @pl.when(core == 0)
        def _():
            pltpu.sync_copy(nwbuf, nw_out.at[pl.ds(tb, NG)])

        _hop(xh_ref, acc1, rowbuf, colbuf, nwbuf, bufs, gsem, ssem, D,
             async_scatter=False)
        plsc.subcore_barrier()
        pltpu.sync_copy(acc1.at[pl.ds(ns, NSLC)],
                        y1.at[pl.ds(roff + ns, NSLC)])
        _zero_acc(acc1, g0, ns, D)
        plsc.subcore_barrier()

        _hop(y1, acc1, rowbuf, colbuf, nwbuf, bufs, gsem, ssem, D,
             async_scatter=False)
        plsc.subcore_barrier()
        pltpu.sync_copy(acc1.at[pl.ds(ns, NSLC)],
                        y2.at[pl.ds(roff + ns, NSLC)])

    return k(xh2, row2, col2, ew2)


def _sc_c(rh2, row2, col2, nw2):
    W = 32

    @functools.partial(
        pl.kernel,
        out_type=[jax.ShapeDtypeStruct((2 * NP, W), f32)] * 2,
        mesh=_MESH,
        scratch_types=[
            pltpu.VMEM((NG, G), i32),
            pltpu.VMEM((NG, G), i32),
            pltpu.VMEM((NG, G), f32),
            pltpu.VMEM((G, W), f32),
            pltpu.VMEM((G, W), f32),
            pltpu.VMEM((G, W), f32),
            pltpu.VMEM((G, W), f32),
            pltpu.VMEM_SHARED((NP, W), f32),
            pltpu.SemaphoreType.DMA,
            pltpu.SemaphoreType.DMA,
        ],
        compiler_params=_PARAMS,
    )
    def k(rh_ref, row_ref, col_ref, nw_ref,
          c1, c2,
          rowbuf, colbuf, nwbuf, g0, g1, s0, s1, acc1, gsem, ssem):
        bufs = (g0, g1, s0, s1)
        core = lax.axis_index("c")
        tile = lax.axis_index("s")
        tb = tile * NG
        ns = tile * NSLC
        roff = core * NP

        pltpu.sync_copy(row_ref.at[pl.ds(tb, NG)], rowbuf)
        pltpu.sync_copy(col_ref.at[pl.ds(tb, NG)], colbuf)
        pltpu.sync_copy(nw_ref.at[pl.ds(tb, NG)], nwbuf)

        def obody(i, c):
            j = i // 8
            q = i % 8
            sl = pl.ds(q * 16, 16)
            rowbuf[j, sl] = rowbuf[j, sl] + roff
            return c

        lax.fori_loop(0, NG * 8, obody, 0)

        _zero_acc(acc1, g0, ns, W)
        plsc.subcore_barrier()

        _hop(rh_ref, acc1, rowbuf, colbuf, nwbuf, bufs, gsem, ssem, W)
        plsc.subcore_barrier()
        pltpu.sync_copy(acc1.at[pl.ds(ns, NSLC)],
                        c1.at[pl.ds(roff + ns, NSLC)])
        _zero_acc(acc1, g0, ns, W)
        plsc.subcore_barrier()

        _hop(c1, acc1, rowbuf, colbuf, nwbuf, bufs, gsem, ssem, W)
        plsc.subcore_barrier()
        pltpu.sync_copy(acc1.at[pl.ds(ns, NSLC)],
                        c2.at[pl.ds(roff + ns, NSLC)])

    return k(rh2, row2, col2, nw2)


BLK1 = 1280
BLK2 = 1000


def _tc1(feats, ws, b):
    def body(x_r, h_r, a2, a3, a4, a5, w0, w1, w2, w3, w4, w5, b_r,
             rha_r, rhb_r, z_r):
        ins = (x_r, h_r, a2, a3, a4, a5)
        wrefs = (w0, w1, w2, w3, w4, w5)
        acc = b_r[:]
        for a, w in zip(ins, wrefs):
            acc = acc + jnp.dot(a[:], w[:], preferred_element_type=f32)
        r = jax.nn.sigmoid(acc[:, :D])
        z = jax.nn.sigmoid(acc[:, D:])
        rh = r * h_r[:]
        rha_r[:] = rh[:, :32]
        rhb_r[:] = rh[:, 32:]
        z_r[:] = z

    grid = (NP // BLK1,)
    fspec = pl.BlockSpec((BLK1, D), lambda i: (i, 0))
    wspec = pl.BlockSpec((D, 2 * D), lambda i: (0, 0))
    bspec = pl.BlockSpec((1, 2 * D), lambda i: (0, 0))
    return pl.pallas_call(
        body,
        grid=grid,
        in_specs=[fspec] * 6 + [wspec] * 6 + [bspec],
        out_specs=[
            pl.BlockSpec((BLK1, 32), lambda i: (i, 0)),
            pl.BlockSpec((BLK1, 32), lambda i: (i, 0)),
            pl.BlockSpec((BLK1, D), lambda i: (i, 0)),
        ],
        out_shape=[
            jax.ShapeDtypeStruct((NP, 32), f32),
            jax.ShapeDtypeStruct((NP, 32), f32),
            jax.ShapeDtypeStruct((NP, D), f32),
        ],
    )(*feats, *ws, b)


def _tc2(feats, widths, ws, b, h, z):
    def body(*refs):
        in_rs = refs[:9]
        h_r, z_r = refs[9], refs[10]
        w_rs = refs[11:20]
        b_r = refs[20]
        out_r = refs[21]
        acc = b_r[:]
        for a, w in zip(in_rs, w_rs):
            acc = acc + jnp.dot(a[:], w[:], preferred_element_type=f32)
        c = jnp.tanh(acc)
        zz = z_r[:]
        out_r[:] = zz * h_r[:] + (1.0 - zz) * c

    grid = (N // BLK2,)
    in_specs = (
        [pl.BlockSpec((BLK2, w), lambda i: (i, 0)) for w in widths]
        + [pl.BlockSpec((BLK2, D), lambda i: (i, 0))] * 2
        + [pl.BlockSpec((w, D), lambda i: (0, 0)) for w in widths]
        + [pl.BlockSpec((1, D), lambda i: (0, 0))]
    )
    return pl.pallas_call(
        body,
        grid=grid,
        in_specs=in_specs,
        out_specs=pl.BlockSpec((BLK2, D), lambda i: (i, 0)),
        out_shape=jax.ShapeDtypeStruct((N, D), f32),
    )(*feats, h, z, *ws, b)


def kernel(x, h, edge_index, edge_weight,
           Wr0, Wr1, Wr2, br, Wz0, Wz1, Wz2, bz, Wc0, Wc1, Wc2, bc):
    row = edge_index[0]
    col = edge_index[1]

    def prep(a, pad_val):
        a2 = a.reshape(NT, EPT)
        pad = jnp.full((NT, EPT_PAD - EPT), pad_val, a.dtype)
        return jnp.concatenate([a2, pad], axis=1).reshape(R2D, G)

    row2 = prep(row, NP - 1)
    col2 = prep(col, NP - 1)
    ew2 = prep(edge_weight, 0.0)
    xp = jnp.pad(x, ((0, NP - N), (0, 0)))
    hp = jnp.pad(h, ((0, NP - N), (0, 0)))
    xh2 = jnp.concatenate([xp, hp], axis=0)

    y1, y2, nw2 = _sc_rz(xh2, row2, col2, ew2)
    y1x, y1h = y1[:NP], y1[NP:]
    y2x, y2h = y2[:NP], y2[NP:]

    sl0, sl1 = slice(0, D), slice(D, 2 * D)
    wrz = [jnp.concatenate([wr[:, sl].T, wz[:, sl].T], axis=1)
           for (wr, wz, sl) in [(Wr0, Wz0, sl0), (Wr0, Wz0, sl1),
                                (Wr1, Wz1, sl0), (Wr1, Wz1, sl1),
                                (Wr2, Wz2, sl0), (Wr2, Wz2, sl1)]]
    brz = jnp.concatenate([br, bz]).reshape(1, 2 * D)
    rha, rhb, z = _tc1((xp, hp, y1x, y1h, y2x, y2h), wrz, brz)

    rh2 = jnp.concatenate([rha, rhb], axis=0)
    c1, c2 = _sc_c(rh2, row2, col2, nw2)
    c1a, c1b = c1[:NP], c1[NP:]
    c2a, c2b = c2[:NP], c2[NP:]

    cfeats = (x, rha, rhb, y1x, c1a, c1b, y2x, c2a, c2b)
    cwidths = [D, 32, 32, D, 32, 32, D, 32, 32]
    cws = [Wc0[:, :64].T, Wc0[:, 64:96].T, Wc0[:, 96:].T,
           Wc1[:, :64].T, Wc1[:, 64:96].T, Wc1[:, 96:].T,
           Wc2[:, :64].T, Wc2[:, 64:96].T, Wc2[:, 96:].T]
    return _tc2(cfeats, cwidths, cws, bc.reshape(1, D), h, z)


# speedup vs baseline: 1.0665x; 1.0665x over previous
import functools

import jax
import jax.numpy as jnp
from jax import lax
from jax.experimental import pallas as pl
from jax.experimental.pallas import tpu as pltpu
from jax.experimental.pallas import tpu_sc as plsc

N = 10000
NP = 10240
E = 320000
D = 64
NT = 16
EPT = E // NT
G = 128
NG = 160
EPT_PAD = NG * G
R2D = NT * NG
NSLC = NP // NT

f32 = jnp.float32
i32 = jnp.int32

_MESH = plsc.VectorSubcoreMesh(core_axis_name="c", subcore_axis_name="s")
_PARAMS = pltpu.CompilerParams(needs_layout_passes=False,
                               use_tc_tiling_on_sc=False)


def _zero_buf(buf, width):
    zero16 = jnp.zeros((16,), f32)

    def body(i, c):
        for q in range(width // 16):
            buf[i, pl.ds(q * 16, 16)] = zero16
        return c

    lax.fori_loop(0, G, body, 0)


def _zero_acc(acc, buf, ns, width):
    _zero_buf(buf, width)

    def body(kk, c):
        pltpu.sync_copy(buf, acc.at[pl.ds(ns + kk * G, G)])
        return c

    lax.fori_loop(0, NSLC // G, body, 0)
    rem = NSLC % G
    if rem:
        pltpu.sync_copy(buf.at[pl.ds(0, rem)],
                        acc.at[pl.ds(ns + (NSLC // G) * G, rem)])


def _scale(gbuf, sbuf, nwbuf, j, width):
    def ebody(e, c):
        bc = plsc.load_gather(
            nwbuf, [jnp.full((16,), j, i32), jnp.full((16,), e, i32)])
        for q in range(width // 16):
            sl = pl.ds(q * 16, 16)
            sbuf[e, sl] = gbuf[e, sl] * bc
        return c

    lax.fori_loop(0, G, ebody, 0)


def _hop(table_ref, acc, rowbuf, colbuf, nwbuf, bufs, gsem, width):
    g0, g1 = bufs
    gb = (g0, g1)

    pltpu.async_copy(table_ref.at[rowbuf.at[0]], g0, gsem)
    pltpu.async_copy(table_ref.at[rowbuf.at[1]], g1, gsem)

    def ibody(i, c):
        for p in range(2):
            j = i * 2 + p
            pltpu.make_async_copy(
                table_ref.at[rowbuf.at[j]], gb[p], gsem).wait()
            _scale(gb[p], gb[p], nwbuf, j, width)
            pltpu.sync_copy(gb[p], acc.at[colbuf.at[j]], add=True)

            @pl.when(j + 2 < NG)
            def _():
                pltpu.async_copy(table_ref.at[rowbuf.at[j + 2]], gb[p], gsem)
        return c

    lax.fori_loop(0, NG // 2, ibody, 0)


def _sc_rz(xh2, row2, col2, ew2):
    @functools.partial(
        pl.kernel,
        out_type=[jax.ShapeDtypeStruct((2 * NP, D), f32)] * 2
        + [jax.ShapeDtypeStruct((R2D, G), f32)],
        mesh=_MESH,
        scratch_types=[
            pltpu.VMEM((NG, G), i32),
            pltpu.VMEM((NG, G), i32),
            pltpu.VMEM((NG, G), f32),
            pltpu.VMEM((NP,), f32),
            pltpu.VMEM((G, D), f32),
            pltpu.VMEM((G, D), f32),
            pltpu.VMEM((G,), f32),
            pltpu.VMEM_SHARED((NP,), f32),
            pltpu.VMEM_SHARED((NP, D), f32),
            pltpu.SemaphoreType.DMA,
        ],
        compiler_params=_PARAMS,
    )
    def k(xh_ref, row_ref, col_ref, ew_ref,
          y1, y2, nw_out,
          rowbuf, colbuf, nwbuf, degbuf, g0, g1, onesbuf,
          deg_acc, acc1, gsem):
        bufs = (g0, g1)
        core = lax.axis_index("c")
        tile = lax.axis_index("s")
        tb = tile * NG
        ns = tile * NSLC
        zero16 = jnp.zeros((16,), f32)

        pltpu.sync_copy(row_ref.at[pl.ds(tb, NG)], rowbuf)
        pltpu.sync_copy(col_ref.at[pl.ds(tb, NG)], colbuf)
        pltpu.sync_copy(ew_ref.at[pl.ds(tb, NG)], nwbuf)

        _zero_acc(acc1, g0, ns, D)

        def zdeg(i, c):
            degbuf[pl.ds(ns + i * 16, 16)] = zero16
            return c

        lax.fori_loop(0, NSLC // 16, zdeg, 0)
        pltpu.sync_copy(degbuf.at[pl.ds(ns, NSLC)], deg_acc.at[pl.ds(ns, NSLC)])

        def ones_(i, c):
            onesbuf[pl.ds(i * 16, 16)] = jnp.ones((16,), f32)
            return c

        lax.fori_loop(0, G // 16, ones_, 0)
        plsc.subcore_barrier()

        def dbody(j, c):
            pltpu.sync_copy(onesbuf, deg_acc.at[rowbuf.at[j]], add=True)
            return c

        lax.fori_loop(0, NG, dbody, 0)
        plsc.subcore_barrier()

        pltpu.sync_copy(deg_acc, degbuf)
        roff = core * NP

        def nbody(i, c):
            j = i // 8
            q = i % 8
            sl = pl.ds(q * 16, 16)
            idx = rowbuf[j, sl]
            dg = jnp.maximum(plsc.load_gather(degbuf, [idx]), 1.0)
            nwbuf[j, sl] = nwbuf[j, sl] / dg
            rowbuf[j, sl] = idx + roff
            return c

        lax.fori_loop(0, NG * 8, nbody, 0)

        @pl.when(core == 0)
        def _():
            pltpu.sync_copy(nwbuf, nw_out.at[pl.ds(tb, NG)])

        _hop(xh_ref, acc1, rowbuf, colbuf, nwbuf, bufs, gsem, D)
        plsc.subcore_barrier()
        pltpu.sync_copy(acc1.at[pl.ds(ns, NSLC)],
                        y1.at[pl.ds(roff + ns, NSLC)])
        _zero_acc(acc1, g0, ns, D)
        plsc.subcore_barrier()

        _hop(y1, acc1, rowbuf, colbuf, nwbuf, bufs, gsem, D)
        plsc.subcore_barrier()
        pltpu.sync_copy(acc1.at[pl.ds(ns, NSLC)],
                        y2.at[pl.ds(roff + ns, NSLC)])

    return k(xh2, row2, col2, ew2)


def _sc_c(rh2, row2, col2, nw2):
    W = 32

    @functools.partial(
        pl.kernel,
        out_type=[jax.ShapeDtypeStruct((2 * NP, W), f32)] * 2,
        mesh=_MESH,
        scratch_types=[
            pltpu.VMEM((NG, G), i32),
            pltpu.VMEM((NG, G), i32),
            pltpu.VMEM((NG, G), f32),
            pltpu.VMEM((G, W), f32),
            pltpu.VMEM((G, W), f32),
            pltpu.VMEM_SHARED((NP, W), f32),
            pltpu.SemaphoreType.DMA,
        ],
        compiler_params=_PARAMS,
    )
    def k(rh_ref, row_ref, col_ref, nw_ref,
          c1, c2,
          rowbuf, colbuf, nwbuf, g0, g1, acc1, gsem):
        bufs = (g0, g1)
        core = lax.axis_index("c")
        tile = lax.axis_index("s")
        tb = tile * NG
        ns = tile * NSLC
        roff = core * NP

        pltpu.sync_copy(row_ref.at[pl.ds(tb, NG)], rowbuf)
        pltpu.sync_copy(col_ref.at[pl.ds(tb, NG)], colbuf)
        pltpu.sync_copy(nw_ref.at[pl.ds(tb, NG)], nwbuf)

        def obody(i, c):
            j = i // 8
            q = i % 8
            sl = pl.ds(q * 16, 16)
            rowbuf[j, sl] = rowbuf[j, sl] + roff
            return c

        lax.fori_loop(0, NG * 8, obody, 0)

        _zero_acc(acc1, g0, ns, W)
        plsc.subcore_barrier()

        _hop(rh_ref, acc1, rowbuf, colbuf, nwbuf, bufs, gsem, W)
        plsc.subcore_barrier()
        pltpu.sync_copy(acc1.at[pl.ds(ns, NSLC)],
                        c1.at[pl.ds(roff + ns, NSLC)])
        _zero_acc(acc1, g0, ns, W)
        plsc.subcore_barrier()

        _hop(c1, acc1, rowbuf, colbuf, nwbuf, bufs, gsem, W)
        plsc.subcore_barrier()
        pltpu.sync_copy(acc1.at[pl.ds(ns, NSLC)],
                        c2.at[pl.ds(roff + ns, NSLC)])

    return k(rh2, row2, col2, nw2)


BLK1 = 1280
BLK2 = 1000


def _tc1(feats, ws, b):
    def body(x_r, h_r, a2, a3, a4, a5, w0, w1, w2, w3, w4, w5, b_r,
             rha_r, rhb_r, z_r):
        ins = (x_r, h_r, a2, a3, a4, a5)
        wrefs = (w0, w1, w2, w3, w4, w5)
        acc = b_r[:]
        for a, w in zip(ins, wrefs):
            acc = acc + jnp.dot(a[:], w[:], preferred_element_type=f32)
        r = jax.nn.sigmoid(acc[:, :D])
        z = jax.nn.sigmoid(acc[:, D:])
        rh = r * h_r[:]
        rha_r[:] = rh[:, :32]
        rhb_r[:] = rh[:, 32:]
        z_r[:] = z

    grid = (NP // BLK1,)
    fspec = pl.BlockSpec((BLK1, D), lambda i: (i, 0))
    wspec = pl.BlockSpec((D, 2 * D), lambda i: (0, 0))
    bspec = pl.BlockSpec((1, 2 * D), lambda i: (0, 0))
    return pl.pallas_call(
        body,
        grid=grid,
        in_specs=[fspec] * 6 + [wspec] * 6 + [bspec],
        out_specs=[
            pl.BlockSpec((BLK1, 32), lambda i: (i, 0)),
            pl.BlockSpec((BLK1, 32), lambda i: (i, 0)),
            pl.BlockSpec((BLK1, D), lambda i: (i, 0)),
        ],
        out_shape=[
            jax.ShapeDtypeStruct((NP, 32), f32),
            jax.ShapeDtypeStruct((NP, 32), f32),
            jax.ShapeDtypeStruct((NP, D), f32),
        ],
    )(*feats, *ws, b)


def _tc2(feats, widths, ws, b, h, z):
    def body(*refs):
        in_rs = refs[:9]
        h_r, z_r = refs[9], refs[10]
        w_rs = refs[11:20]
        b_r = refs[20]
        out_r = refs[21]
        acc = b_r[:]
        for a, w in zip(in_rs, w_rs):
            acc = acc + jnp.dot(a[:], w[:], preferred_element_type=f32)
        c = jnp.tanh(acc)
        zz = z_r[:]
        out_r[:] = zz * h_r[:] + (1.0 - zz) * c

    grid = (N // BLK2,)
    in_specs = (
        [pl.BlockSpec((BLK2, w), lambda i: (i, 0)) for w in widths]
        + [pl.BlockSpec((BLK2, D), lambda i: (i, 0))] * 2
        + [pl.BlockSpec((w, D), lambda i: (0, 0)) for w in widths]
        + [pl.BlockSpec((1, D), lambda i: (0, 0))]
    )
    return pl.pallas_call(
        body,
        grid=grid,
        in_specs=in_specs,
        out_specs=pl.BlockSpec((BLK2, D), lambda i: (i, 0)),
        out_shape=jax.ShapeDtypeStruct((N, D), f32),
    )(*feats, h, z, *ws, b)


def kernel(x, h, edge_index, edge_weight,
           Wr0, Wr1, Wr2, br, Wz0, Wz1, Wz2, bz, Wc0, Wc1, Wc2, bc):
    row = edge_index[0]
    col = edge_index[1]

    def prep(a, pad_val):
        a2 = a.reshape(NT, EPT)
        pad = jnp.full((NT, EPT_PAD - EPT), pad_val, a.dtype)
        return jnp.concatenate([a2, pad], axis=1).reshape(R2D, G)

    row2 = prep(row, NP - 1)
    col2 = prep(col, NP - 1)
    ew2 = prep(edge_weight, 0.0)
    xp = jnp.pad(x, ((0, NP - N), (0, 0)))
    hp = jnp.pad(h, ((0, NP - N), (0, 0)))
    xh2 = jnp.concatenate([xp, hp], axis=0)

    y1, y2, nw2 = _sc_rz(xh2, row2, col2, ew2)
    y1x, y1h = y1[:NP], y1[NP:]
    y2x, y2h = y2[:NP], y2[NP:]

    sl0, sl1 = slice(0, D), slice(D, 2 * D)
    wrz = [jnp.concatenate([wr[:, sl].T, wz[:, sl].T], axis=1)
           for (wr, wz, sl) in [(Wr0, Wz0, sl0), (Wr0, Wz0, sl1),
                                (Wr1, Wz1, sl0), (Wr1, Wz1, sl1),
                                (Wr2, Wz2, sl0), (Wr2, Wz2, sl1)]]
    brz = jnp.concatenate([br, bz]).reshape(1, 2 * D)
    rha, rhb, z = _tc1((xp, hp, y1x, y1h, y2x, y2h), wrz, brz)

    rh2 = jnp.concatenate([rha, rhb], axis=0)
    c1, c2 = _sc_c(rh2, row2, col2, nw2)
    c1a, c1b = c1[:NP], c1[NP:]
    c2a, c2b = c2[:NP], c2[NP:]

    cfeats = (x, rha, rhb, y1x, c1a, c1b, y2x, c2a, c2b)
    cwidths = [D, 32, 32, D, 32, 32, D, 32, 32]
    cws = [Wc0[:, :64].T, Wc0[:, 64:96].T, Wc0[:, 96:].T,
           Wc1[:, :64].T, Wc1[:, 64:96].T, Wc1[:, 96:].T,
           Wc2[:, :64].T, Wc2[:, 64:96].T, Wc2[:, 96:].T]
    return _tc2(cfeats, cwidths, cws, bc.reshape(1, D), h, z)


# speedup vs baseline: 1.1129x; 1.0435x over previous
import functools

import jax
import jax.numpy as jnp
from jax import lax
from jax.experimental import pallas as pl
from jax.experimental.pallas import tpu as pltpu
from jax.experimental.pallas import tpu_sc as plsc

N = 10000
NP = 10240
E = 320000
D = 64
NT = 16
EPT = E // NT
G = 128
NG = 160
EPT_PAD = NG * G
R2D = NT * NG
NSLC = NP // NT

f32 = jnp.float32
i32 = jnp.int32

_MESH = plsc.VectorSubcoreMesh(core_axis_name="c", subcore_axis_name="s")
_PARAMS = pltpu.CompilerParams(needs_layout_passes=False,
                               use_tc_tiling_on_sc=False)


def _zero_buf(buf, width):
    zero16 = jnp.zeros((16,), f32)

    def body(i, c):
        for q in range(width // 16):
            buf[i, pl.ds(q * 16, 16)] = zero16
        return c

    lax.fori_loop(0, G, body, 0)


def _zero_acc(acc, buf, ns, width):
    _zero_buf(buf, width)

    def body(kk, c):
        pltpu.sync_copy(buf, acc.at[pl.ds(ns + kk * G, G)])
        return c

    lax.fori_loop(0, NSLC // G, body, 0)
    rem = NSLC % G
    if rem:
        pltpu.sync_copy(buf.at[pl.ds(0, rem)],
                        acc.at[pl.ds(ns + (NSLC // G) * G, rem)])


def _scale(gbuf, sbuf, nwbuf, j, width):
    def ebody(e, c):
        bc = plsc.load_gather(
            nwbuf, [jnp.full((16,), j, i32), jnp.full((16,), e, i32)])
        for q in range(width // 16):
            sl = pl.ds(q * 16, 16)
            sbuf[e, sl] = gbuf[e, sl] * bc
        return c

    lax.fori_loop(0, G, ebody, 0)


def _hop2(table_ref, acc, rowbuf, colbuf, nwbuf, bufs, gsem, width):
    g0, g1 = bufs
    gb = (g0, g1)

    pltpu.async_copy(table_ref.at[rowbuf.at[0]], g0, gsem)
    pltpu.async_copy(table_ref.at[rowbuf.at[1]], g1, gsem)

    def ibody(i, c):
        for p in range(2):
            j = i * 2 + p
            pltpu.make_async_copy(
                table_ref.at[rowbuf.at[j]], gb[p], gsem).wait()
            _scale(gb[p], gb[p], nwbuf, j, width)
            pltpu.sync_copy(gb[p], acc.at[colbuf.at[j]], add=True)

            @pl.when(j + 2 < NG)
            def _():
                pltpu.async_copy(table_ref.at[rowbuf.at[j + 2]], gb[p], gsem)
        return c

    lax.fori_loop(0, NG // 2, ibody, 0)


def _hop(table_ref, acc, rowbuf, colbuf, nwbuf, bufs, gsem, ssem, width):
    gb = bufs

    pltpu.async_copy(table_ref.at[rowbuf.at[0]], gb[0], gsem)
    pltpu.async_copy(table_ref.at[rowbuf.at[1]], gb[1], gsem)

    def ibody(i, c):
        for p in range(4):
            j = i * 4 + p
            pltpu.make_async_copy(
                table_ref.at[rowbuf.at[j]], gb[p], gsem).wait()
            _scale(gb[p], gb[p], nwbuf, j, width)
            pltpu.async_copy(gb[p], acc.at[colbuf.at[j]], ssem, add=True)

            @pl.when(j + 2 < NG)
            def _():
                jm2 = jnp.maximum(j - 2, 0)

                @pl.when(j >= 2)
                def _():
                    pltpu.make_async_copy(
                        gb[(p + 2) % 4], acc.at[colbuf.at[jm2]], ssem).wait()

                pltpu.async_copy(
                    table_ref.at[rowbuf.at[j + 2]], gb[(p + 2) % 4], gsem)
        return c

    lax.fori_loop(0, NG // 4, ibody, 0)
    for t in range(4):
        j = NG - 4 + t
        pltpu.make_async_copy(gb[j % 4], acc.at[colbuf.at[j]], ssem).wait()


def _sc_rz(xh2, row2, col2, ew2):
    @functools.partial(
        pl.kernel,
        out_type=[jax.ShapeDtypeStruct((2 * NP, D), f32)] * 2
        + [jax.ShapeDtypeStruct((R2D, G), f32)],
        mesh=_MESH,
        scratch_types=[
            pltpu.VMEM((NG, G), i32),
            pltpu.VMEM((NG, G), i32),
            pltpu.VMEM((NG, G), f32),
            pltpu.VMEM((NP,), f32),
            pltpu.VMEM((G, D), f32),
            pltpu.VMEM((G, D), f32),
            pltpu.VMEM((G,), f32),
            pltpu.VMEM_SHARED((NP,), f32),
            pltpu.VMEM_SHARED((NP, D), f32),
            pltpu.SemaphoreType.DMA,
        ],
        compiler_params=_PARAMS,
    )
    def k(xh_ref, row_ref, col_ref, ew_ref,
          y1, y2, nw_out,
          rowbuf, colbuf, nwbuf, degbuf, g0, g1, onesbuf,
          deg_acc, acc1, gsem):
        bufs = (g0, g1)
        core = lax.axis_index("c")
        tile = lax.axis_index("s")
        tb = tile * NG
        ns = tile * NSLC
        zero16 = jnp.zeros((16,), f32)

        pltpu.sync_copy(row_ref.at[pl.ds(tb, NG)], rowbuf)
        pltpu.sync_copy(col_ref.at[pl.ds(tb, NG)], colbuf)
        pltpu.sync_copy(ew_ref.at[pl.ds(tb, NG)], nwbuf)

        _zero_acc(acc1, g0, ns, D)

        def zdeg(i, c):
            degbuf[pl.ds(ns + i * 16, 16)] = zero16
            return c

        lax.fori_loop(0, NSLC // 16, zdeg, 0)
        pltpu.sync_copy(degbuf.at[pl.ds(ns, NSLC)], deg_acc.at[pl.ds(ns, NSLC)])

        def ones_(i, c):
            onesbuf[pl.ds(i * 16, 16)] = jnp.ones((16,), f32)
            return c

        lax.fori_loop(0, G // 16, ones_, 0)
        plsc.subcore_barrier()

        def dbody(j, c):
            pltpu.sync_copy(onesbuf, deg_acc.at[rowbuf.at[j]], add=True)
            return c

        lax.fori_loop(0, NG, dbody, 0)
        plsc.subcore_barrier()

        pltpu.sync_copy(deg_acc, degbuf)
        roff = core * NP

        def nbody(i, c):
            j = i // 8
            q = i % 8
            sl = pl.ds(q * 16, 16)
            idx = rowbuf[j, sl]
            dg = jnp.maximum(plsc.load_gather(degbuf, [idx]), 1.0)
            nwbuf[j, sl] = nwbuf[j, sl] / dg
            rowbuf[j, sl] = idx + roff
            return c

        lax.fori_loop(0, NG * 8, nbody, 0)

        @pl.when(core == 0)
        def _():
            pltpu.sync_copy(nwbuf, nw_out.at[pl.ds(tb, NG)])

        _hop2(xh_ref, acc1, rowbuf, colbuf, nwbuf, bufs, gsem, D)
        plsc.subcore_barrier()
        pltpu.sync_copy(acc1.at[pl.ds(ns, NSLC)],
                        y1.at[pl.ds(roff + ns, NSLC)])
        _zero_acc(acc1, g0, ns, D)
        plsc.subcore_barrier()

        _hop2(y1, acc1, rowbuf, colbuf, nwbuf, bufs, gsem, D)
        plsc.subcore_barrier()
        pltpu.sync_copy(acc1.at[pl.ds(ns, NSLC)],
                        y2.at[pl.ds(roff + ns, NSLC)])

    return k(xh2, row2, col2, ew2)


def _sc_c(rh2, row2, col2, nw2):
    W = 32

    @functools.partial(
        pl.kernel,
        out_type=[jax.ShapeDtypeStruct((2 * NP, W), f32)] * 2,
        mesh=_MESH,
        scratch_types=[
            pltpu.VMEM((NG, G), i32),
            pltpu.VMEM((NG, G), i32),
            pltpu.VMEM((NG, G), f32),
            pltpu.VMEM((G, W), f32),
            pltpu.VMEM((G, W), f32),
            pltpu.VMEM((G, W), f32),
            pltpu.VMEM((G, W), f32),
            pltpu.VMEM_SHARED((NP, W), f32),
            pltpu.SemaphoreType.DMA,
            pltpu.SemaphoreType.DMA,
        ],
        compiler_params=_PARAMS,
    )
    def k(rh_ref, row_ref, col_ref, nw_ref,
          c1, c2,
          rowbuf, colbuf, nwbuf, g0, g1, g2, g3, acc1, gsem, ssem):
        bufs = (g0, g1, g2, g3)
        core = lax.axis_index("c")
        tile = lax.axis_index("s")
        tb = tile * NG
        ns = tile * NSLC
        roff = core * NP

        pltpu.sync_copy(row_ref.at[pl.ds(tb, NG)], rowbuf)
        pltpu.sync_copy(col_ref.at[pl.ds(tb, NG)], colbuf)
        pltpu.sync_copy(nw_ref.at[pl.ds(tb, NG)], nwbuf)

        def obody(i, c):
            j = i // 8
            q = i % 8
            sl = pl.ds(q * 16, 16)
            rowbuf[j, sl] = rowbuf[j, sl] + roff
            return c

        lax.fori_loop(0, NG * 8, obody, 0)

        _zero_acc(acc1, g0, ns, W)
        plsc.subcore_barrier()

        _hop(rh_ref, acc1, rowbuf, colbuf, nwbuf, bufs, gsem, ssem, W)
        plsc.subcore_barrier()
        pltpu.sync_copy(acc1.at[pl.ds(ns, NSLC)],
                        c1.at[pl.ds(roff + ns, NSLC)])
        _zero_acc(acc1, g0, ns, W)
        plsc.subcore_barrier()

        _hop(c1, acc1, rowbuf, colbuf, nwbuf, bufs, gsem, ssem, W)
        plsc.subcore_barrier()
        pltpu.sync_copy(acc1.at[pl.ds(ns, NSLC)],
                        c2.at[pl.ds(roff + ns, NSLC)])

    return k(rh2, row2, col2, nw2)


BLK1 = 1280
BLK2 = 1000


def _tc1(feats, ws, b):
    def body(x_r, h_r, a2, a3, a4, a5, w0, w1, w2, w3, w4, w5, b_r,
             rha_r, rhb_r, z_r):
        ins = (x_r, h_r, a2, a3, a4, a5)
        wrefs = (w0, w1, w2, w3, w4, w5)
        acc = b_r[:]
        for a, w in zip(ins, wrefs):
            acc = acc + jnp.dot(a[:], w[:], preferred_element_type=f32)
        r = jax.nn.sigmoid(acc[:, :D])
        z = jax.nn.sigmoid(acc[:, D:])
        rh = r * h_r[:]
        rha_r[:] = rh[:, :32]
        rhb_r[:] = rh[:, 32:]
        z_r[:] = z

    grid = (NP // BLK1,)
    fspec = pl.BlockSpec((BLK1, D), lambda i: (i, 0))
    wspec = pl.BlockSpec((D, 2 * D), lambda i: (0, 0))
    bspec = pl.BlockSpec((1, 2 * D), lambda i: (0, 0))
    return pl.pallas_call(
        body,
        grid=grid,
        in_specs=[fspec] * 6 + [wspec] * 6 + [bspec],
        out_specs=[
            pl.BlockSpec((BLK1, 32), lambda i: (i, 0)),
            pl.BlockSpec((BLK1, 32), lambda i: (i, 0)),
            pl.BlockSpec((BLK1, D), lambda i: (i, 0)),
        ],
        out_shape=[
            jax.ShapeDtypeStruct((NP, 32), f32),
            jax.ShapeDtypeStruct((NP, 32), f32),
            jax.ShapeDtypeStruct((NP, D), f32),
        ],
    )(*feats, *ws, b)


def _tc2(feats, widths, ws, b, h, z):
    def body(*refs):
        in_rs = refs[:9]
        h_r, z_r = refs[9], refs[10]
        w_rs = refs[11:20]
        b_r = refs[20]
        out_r = refs[21]
        acc = b_r[:]
        for a, w in zip(in_rs, w_rs):
            acc = acc + jnp.dot(a[:], w[:], preferred_element_type=f32)
        c = jnp.tanh(acc)
        zz = z_r[:]
        out_r[:] = zz * h_r[:] + (1.0 - zz) * c

    grid = (N // BLK2,)
    in_specs = (
        [pl.BlockSpec((BLK2, w), lambda i: (i, 0)) for w in widths]
        + [pl.BlockSpec((BLK2, D), lambda i: (i, 0))] * 2
        + [pl.BlockSpec((w, D), lambda i: (0, 0)) for w in widths]
        + [pl.BlockSpec((1, D), lambda i: (0, 0))]
    )
    return pl.pallas_call(
        body,
        grid=grid,
        in_specs=in_specs,
        out_specs=pl.BlockSpec((BLK2, D), lambda i: (i, 0)),
        out_shape=jax.ShapeDtypeStruct((N, D), f32),
    )(*feats, h, z, *ws, b)


def kernel(x, h, edge_index, edge_weight,
           Wr0, Wr1, Wr2, br, Wz0, Wz1, Wz2, bz, Wc0, Wc1, Wc2, bc):
    row = edge_index[0]
    col = edge_index[1]

    def prep(a, pad_val):
        a2 = a.reshape(NT, EPT)
        pad = jnp.full((NT, EPT_PAD - EPT), pad_val, a.dtype)
        return jnp.concatenate([a2, pad], axis=1).reshape(R2D, G)

    row2 = prep(row, NP - 1)
    col2 = prep(col, NP - 1)
    ew2 = prep(edge_weight, 0.0)
    xp = jnp.pad(x, ((0, NP - N), (0, 0)))
    hp = jnp.pad(h, ((0, NP - N), (0, 0)))
    xh2 = jnp.concatenate([xp, hp], axis=0)

    y1, y2, nw2 = _sc_rz(xh2, row2, col2, ew2)
    y1x, y1h = y1[:NP], y1[NP:]
    y2x, y2h = y2[:NP], y2[NP:]

    sl0, sl1 = slice(0, D), slice(D, 2 * D)
    wrz = [jnp.concatenate([wr[:, sl].T, wz[:, sl].T], axis=1)
           for (wr, wz, sl) in [(Wr0, Wz0, sl0), (Wr0, Wz0, sl1),
                                (Wr1, Wz1, sl0), (Wr1, Wz1, sl1),
                                (Wr2, Wz2, sl0), (Wr2, Wz2, sl1)]]
    brz = jnp.concatenate([br, bz]).reshape(1, 2 * D)
    rha, rhb, z = _tc1((xp, hp, y1x, y1h, y2x, y2h), wrz, brz)

    rh2 = jnp.concatenate([rha, rhb], axis=0)
    c1, c2 = _sc_c(rh2, row2, col2, nw2)
    c1a, c1b = c1[:NP], c1[NP:]
    c2a, c2b = c2[:NP], c2[NP:]

    cfeats = (x, rha, rhb, y1x, c1a, c1b, y2x, c2a, c2b)
    cwidths = [D, 32, 32, D, 32, 32, D, 32, 32]
    cws = [Wc0[:, :64].T, Wc0[:, 64:96].T, Wc0[:, 96:].T,
           Wc1[:, :64].T, Wc1[:, 64:96].T, Wc1[:, 96:].T,
           Wc2[:, :64].T, Wc2[:, 64:96].T, Wc2[:, 96:].T]
    return _tc2(cfeats, cwidths, cws, bc.reshape(1, D), h, z)
